# Initial kernel scaffold; baseline (speedup 1.0000x reference)
import jax
import jax.numpy as jnp
from jax.experimental import pallas as pl


def kernel(features, h3_nodes, graph_edge_index, graph_edge_attr, latent_edge_index, latent_edge_attr, params):
    raise NotImplementedError("write your pallas kernel here")


# speedup vs baseline: 1.3739x; 1.3739x over previous
import functools

import jax
import jax.numpy as jnp
from jax import lax
from jax.experimental import pallas as pl
from jax.experimental.pallas import tpu as pltpu
from jax.experimental.pallas import tpu_sc as plsc

_N_LL = 16200
_N_H3 = 5882
_D = 256
_NC, _NS = 2, 16
_NW = _NC * _NS
_E_PAD = 16384
_EPW = _E_PAD // _NW
_CH = 128
_NCHUNK = _EPW // _CH
_A_PAD = 5888
_CHE = 512
_EPC = _E_PAD // _NC


def _row_spec(dim, rows):
    return pl.BlockSpec((rows, dim), lambda i: (i, 0))


def _fused_mlp(xs, w1s, b1, w2, b2, w3, b3, gamma, beta, nrows,
               residual=None, block_rows=512, transpose_out=False):
    xs = [x if isinstance(x, list) else [x] for x in xs]
    flat = [t for grp in xs for t in grp]
    sizes = [len(grp) for grp in xs]
    has_res = residual is not None
    R = block_rows

    def body(*refs):
        o_ref = refs[-1]
        nfx = len(flat)
        vals = []
        for t, r in zip(flat, refs[:nfx]):
            if isinstance(t, tuple):
                vals.append(r[...][0])
            else:
                vals.append(r[...])
        w1r = refs[nfx:nfx + len(w1s)]
        b1r, w2r, b2r, w3r, b3r, gr, ber = refs[nfx + len(w1s):nfx + len(w1s) + 7]
        res_ref = refs[nfx + len(w1s) + 7] if has_res else None

        h = None
        pos = 0
        for k, sz in enumerate(sizes):
            xk = vals[pos]
            for t in range(1, sz):
                xk = xk + vals[pos + t]
            transposed = isinstance(flat[pos], tuple)
            pos += sz
            if transposed:
                term = lax.dot_general(
                    xk, w1r[k][...], (((0,), (0,)), ((), ())),
                    preferred_element_type=jnp.float32)
            else:
                term = jnp.dot(xk, w1r[k][...],
                               preferred_element_type=jnp.float32)
            h = term if h is None else h + term
        h = h + b1r[...]
        h = h * jax.nn.sigmoid(h)
        h = jnp.dot(h, w2r[...], preferred_element_type=jnp.float32) + b2r[...]
        h = h * jax.nn.sigmoid(h)
        y = jnp.dot(h, w3r[...], preferred_element_type=jnp.float32) + b3r[...]
        mu = jnp.mean(y, axis=-1, keepdims=True)
        var = jnp.mean((y - mu) ** 2, axis=-1, keepdims=True)
        y = (y - mu) * lax.rsqrt(var + 1e-5) * gr[...] + ber[...]
        if has_res:
            y = y + res_ref[...]
        o_ref[...] = y.T if transpose_out else y

    in_specs = []
    in_arrays = []
    for t in flat:
        if isinstance(t, tuple):
            _, arr, lead = t
            in_specs.append(pl.BlockSpec(
                (1, arr.shape[1], R),
                functools.partial(lambda lead_, i: (lead_, 0, i), lead)))
            in_arrays.append(arr)
        else:
            in_specs.append(_row_spec(t.shape[-1], R))
            in_arrays.append(t)
    for w in w1s:
        in_specs.append(pl.BlockSpec(w.shape, lambda i: (0, 0)))
        in_arrays.append(w)
    for a in (b1, w2, b2, w3, b3, gamma, beta):
        in_specs.append(pl.BlockSpec(a.shape, (lambda i: (0, 0)) if a.ndim == 2
                                     else (lambda i: (0,))))
        in_arrays.append(a)
    if has_res:
        in_specs.append(_row_spec(residual.shape[-1], R))
        in_arrays.append(residual)

    if transpose_out:
        out_spec = pl.BlockSpec((_D, R), lambda i: (0, i))
        out_shape = jax.ShapeDtypeStruct((_D, nrows), jnp.float32)
    else:
        out_spec = _row_spec(_D, R)
        out_shape = jax.ShapeDtypeStruct((nrows, _D), jnp.float32)

    return pl.pallas_call(
        body,
        grid=(pl.cdiv(nrows, R),),
        in_specs=in_specs,
        out_specs=out_spec,
        out_shape=out_shape,
    )(*in_arrays)


def _sc_mesh():
    return plsc.VectorSubcoreMesh(core_axis_name="c", subcore_axis_name="s",
                                  num_cores=_NC, num_subcores=_NS)


def _sc_gather(table, idx):
    @functools.partial(
        pl.kernel,
        out_type=jax.ShapeDtypeStruct((_E_PAD, _D), jnp.float32),
        mesh=_sc_mesh(),
        scratch_types=[
            pltpu.VMEM((_CH,), jnp.int32),
            pltpu.VMEM((_CH, _D), jnp.float32),
            pltpu.SemaphoreType.DMA,
        ],
    )
    def k(table_hbm, idx_hbm, out_hbm, idx_v, rows_v, sem):
        wid = lax.axis_index("s") * _NC + lax.axis_index("c")
        base = wid * _EPW
        for j in range(_NCHUNK):
            off = base + j * _CH
            pltpu.sync_copy(idx_hbm.at[pl.ds(off, _CH)], idx_v)
            pltpu.async_copy(table_hbm.at[idx_v], rows_v, sem).wait()
            pltpu.sync_copy(rows_v, out_hbm.at[pl.ds(off, _CH)])

    return k(table, idx)


def _sc_segment_sum(et, cidx, zeros):
    @functools.partial(
        pl.kernel,
        out_type=jax.ShapeDtypeStruct((_NC, _D, _A_PAD), jnp.float32),
        mesh=_sc_mesh(),
        compiler_params=pltpu.CompilerParams(needs_layout_passes=False),
        scratch_types=[
            pltpu.VMEM((_CHE,), jnp.int32),
            pltpu.VMEM((16, _CHE), jnp.float32),
            pltpu.VMEM((16, _A_PAD), jnp.float32),
        ],
    )
    def k(et_hbm, cidx_hbm, zeros_hbm, out_hbm, cidx_v, chunk_v, acc_v):
        c = lax.axis_index("c")
        s = lax.axis_index("s")
        pltpu.sync_copy(zeros_hbm, acc_v)
        lane = lax.iota(jnp.int32, 16)
        zero16 = jnp.zeros((16,), jnp.int32)
        base = c * _EPC
        for ch in range(_EPC // _CHE):
            off = base + ch * _CHE
            pltpu.sync_copy(cidx_hbm.at[pl.ds(off, _CHE)], cidx_v)
            pltpu.sync_copy(et_hbm.at[pl.ds(s * 16, 16), pl.ds(off, _CHE)],
                            chunk_v)

            def body(g, _):
                cvec = cidx_v[pl.ds(g * 16, 16)]
                for j in range(16):
                    bucket = jnp.take(cvec, zero16 + j)
                    col = zero16 + (g * 16 + j)
                    val = plsc.load_gather(chunk_v, [lane, col])
                    plsc.addupdate_scatter(acc_v, [lane, bucket], val)
                return 0

            lax.fori_loop(0, _CHE // 16, body, 0)
        pltpu.sync_copy(acc_v, out_hbm.at[c, pl.ds(s * 16, 16)])

    return k(et, cidx, zeros)


def kernel(features, h3_nodes, graph_edge_index, graph_edge_attr,
           latent_edge_index, latent_edge_attr, params):
    feats = features.reshape(-1, features.shape[-1])

    def mlp_params(name):
        (w1, b1), (w2, b2), (w3, b3) = params[name]["layers"]
        g, be = params[name]["ln"]
        return w1, b1, w2, b2, w3, b3, g, be

    w1n, b1n, w2n, b2n, w3n, b3n, gn, ben = mlp_params("node_encoder")
    out_ll = _fused_mlp([feats], [w1n], b1n, w2n, b2n, w3n, b3n, gn, ben,
                        nrows=_N_LL)
    out_h3 = _fused_mlp([h3_nodes], [w1n], b1n, w2n, b2n, w3n, b3n, gn, ben,
                        nrows=_N_H3)

    w1e, b1e, w2e, b2e, w3e, b3e, ge, bee = mlp_params("edge_encoder")
    ea = _fused_mlp([graph_edge_attr], [w1e], b1e, w2e, b2e, w3e, b3e, ge, bee,
                    nrows=_N_LL)

    w1l, b1l, w2l, b2l, w3l, b3l, gl, bel = mlp_params("latent_edge_encoder")
    lat_ea = _fused_mlp([latent_edge_attr], [w1l], b1l, w2l, b2l, w3l, b3l,
                        gl, bel, nrows=latent_edge_attr.shape[0])

    cell = graph_edge_index[1] - _N_LL
    pad_i = jnp.zeros((_E_PAD - _N_LL,), jnp.int32)
    g_rows = _sc_gather(out_h3, jnp.concatenate([cell, pad_i]))

    w1p, b1p, w2p, b2p, w3p, b3p, gp, bep = mlp_params("proc_edge")
    e_new_t = _fused_mlp([out_ll, g_rows, ea],
                         [w1p[:_D], w1p[_D:2 * _D], w1p[2 * _D:]],
                         b1p, w2p, b2p, w3p, b3p, gp, bep,
                         nrows=_E_PAD, residual=ea, transpose_out=True)

    ar = jnp.arange(_E_PAD - _N_LL, dtype=jnp.int32)
    cidx = jnp.concatenate([cell, _N_H3 + (ar % (_A_PAD - _N_H3))])
    zeros = jnp.zeros((16, _A_PAD), jnp.float32)
    parts = _sc_segment_sum(e_new_t, cidx, zeros)

    w1q, b1q, w2q, b2q, w3q, b3q, gq, beq = mlp_params("proc_node")
    out2 = _fused_mlp([out_h3, [("T3", parts, 0), ("T3", parts, 1)]],
                      [w1q[:_D], w1q[_D:]],
                      b1q, w2q, b2q, w3q, b3q, gq, beq,
                      nrows=_N_H3, residual=out_h3)

    return out2, latent_edge_index, lat_ea
